# baseline (speedup 1.0000x reference)
import functools

import jax
import jax.numpy as jnp
from jax import lax
from jax.experimental import pallas as pl
from jax.experimental.pallas import tpu as pltpu
from jax.experimental.pallas import tpu_sc as plsc

_NUM_EXPERTS = 64
_EPAD = 128
_HIDDEN = 2048
_TOP_K = 8
_TOKENS = 16384
_BT = 512

_NEG_INF = float("-inf")

_NW = 32
_TPW = _TOKENS // _NW


def _tree_row_sumsq(x):
    x2 = x * x
    parts = [x2[:, k * 128:(k + 1) * 128] for k in range(x.shape[1] // 128)]
    while len(parts) > 1:
        parts = [parts[2 * j] + parts[2 * j + 1]
                 for j in range(len(parts) // 2)]
    return jnp.sum(parts[0], axis=1, keepdims=True)


def _proto_norm_block(pt_ref, pnt_ref):
    pt = pt_ref[...]
    pnorm = jnp.sqrt(jnp.sum(pt * pt, axis=0, keepdims=True))
    pnt_ref[...] = pt / jnp.maximum(pnorm, 1e-12)


def _probs_block(h_ref, pn_ref, p_ref):
    h = h_ref[...]
    hnorm = jnp.maximum(jnp.sqrt(_tree_row_sumsq(h)), 1e-12)
    hn = h / hnorm
    logits = jax.lax.dot_general(
        hn, pn_ref[...],
        (((1,), (0,)), ((), ())),
        preferred_element_type=jnp.float32,
    )
    iota_f = jax.lax.broadcasted_iota(
        jnp.int32, (_BT, _EPAD), 1).astype(jnp.float32)
    logits = jnp.where(iota_f < _NUM_EXPERTS, logits, _NEG_INF)
    m = jnp.max(logits, axis=1, keepdims=True)
    e = jnp.exp(logits - m)
    z = jnp.sum(e, axis=1, keepdims=True)
    probs = e / z
    p_ref[...] = probs[:, :_NUM_EXPERTS]


def _sc_merge(ak, av, bk, bv):
    rbk = lax.rev(bk, (0,))
    rbv = lax.rev(bv, (0,))
    take_a = (ak > rbk) | ((ak == rbk) & (av < rbv))
    mk = jnp.where(take_a, ak, rbk)
    mv = jnp.where(take_a, av, rbv)
    return plsc.sort_key_val(mk, mv, descending=True)


@functools.partial(
    pl.kernel,
    mesh=plsc.VectorSubcoreMesh(core_axis_name="c", subcore_axis_name="s"),
    out_type=[
        jax.ShapeDtypeStruct((_TOKENS * 16,), jnp.float32),
        jax.ShapeDtypeStruct((_TOKENS * 16,), jnp.int32),
    ],
    scratch_types=[
        pltpu.VMEM((_TPW * _NUM_EXPERTS,), jnp.float32),
        pltpu.VMEM((_TPW * 16,), jnp.float32),
        pltpu.VMEM((_TPW * 16,), jnp.int32),
    ],
    compiler_params=pltpu.CompilerParams(needs_layout_passes=False),
)
def _sc_topk(probs_hbm, w_hbm, i_hbm, pv, wv, iv):
    wid = lax.axis_index("s") * 2 + lax.axis_index("c")
    base = wid * _TPW
    pltpu.sync_copy(probs_hbm.at[pl.ds(base * _NUM_EXPERTS,
                                       _TPW * _NUM_EXPERTS)], pv)

    def body(t, carry):
        ks, vs = [], []
        for g in range(4):
            k = pv[pl.ds(t * _NUM_EXPERTS + g * 16, 16)]
            idx = lax.iota(jnp.int32, 16) + (g * 16)
            k, v = plsc.sort_key_val(k, idx, descending=True)
            ks.append(k)
            vs.append(v)
        k01, v01 = _sc_merge(ks[0], vs[0], ks[1], vs[1])
        k23, v23 = _sc_merge(ks[2], vs[2], ks[3], vs[3])
        kf, vf = _sc_merge(k01, v01, k23, v23)
        wv[pl.ds(t * 16, 16)] = kf
        iv[pl.ds(t * 16, 16)] = vf
        return carry

    lax.fori_loop(0, _TPW, body, 0)
    pltpu.sync_copy(wv, w_hbm.at[pl.ds(base * 16, _TPW * 16)])
    pltpu.sync_copy(iv, i_hbm.at[pl.ds(base * 16, _TPW * 16)])


@jax.jit
def kernel(hidden_states, proto):
    proto_t = jnp.pad(proto, ((0, _EPAD - _NUM_EXPERTS), (0, 0))).T
    pn = pl.pallas_call(
        _proto_norm_block,
        out_shape=jax.ShapeDtypeStruct((_HIDDEN, _EPAD), jnp.float32),
    )(proto_t)

    grid = _TOKENS // _BT
    probs = pl.pallas_call(
        _probs_block,
        grid=(grid,),
        in_specs=[
            pl.BlockSpec((_BT, _HIDDEN), lambda i: (i, 0)),
            pl.BlockSpec((_HIDDEN, _EPAD), lambda i: (0, 0)),
        ],
        out_specs=pl.BlockSpec((_BT, _NUM_EXPERTS), lambda i: (i, 0)),
        out_shape=jax.ShapeDtypeStruct((_TOKENS, _NUM_EXPERTS), jnp.float32),
        compiler_params=pltpu.CompilerParams(
            dimension_semantics=("parallel",),
        ),
    )(hidden_states, pn)

    w1, i1 = _sc_topk(probs.reshape(-1))
    w16 = w1.reshape(_TOKENS, 16)
    i16 = i1.reshape(_TOKENS, 16)
    return w16[:, :_TOP_K], i16[:, :_TOP_K]
